# Initial kernel scaffold; baseline (speedup 1.0000x reference)
import jax
import jax.numpy as jnp
from jax.experimental import pallas as pl


def kernel(nf, edge_index, traj, W_e, b_e, W_n, b_n):
    raise NotImplementedError("write your pallas kernel here")


# speedup vs baseline: 3.0681x; 3.0681x over previous
import functools

import jax
import jax.numpy as jnp
from jax import lax
from jax.experimental import pallas as pl
from jax.experimental.pallas import tpu as pltpu
from jax.experimental.pallas import tpu_sc as plsc

N_NODES = 10000
N_EDGES = 160000
D = 256
F = 128
NC = 2
NS = 16
L = 16
EPW = N_EDGES // NS
K = 80
NCHUNK = EPW // K
ROWS_PER_CP = 200
NROW_CHUNKS = N_NODES // ROWS_PER_CP
NCP_ITERS = -(-NROW_CHUNKS // NS)

RB = 1000
GRID = N_NODES // RB


def _stage1_body(nf_ref, we0_ref, we1_ref, be_ref, wn0_ref, bn_ref,
                 a2_ref, b2_ref, p_ref):
    x = nf_ref[...]
    a = jnp.dot(x, we0_ref[...], preferred_element_type=jnp.float32) + be_ref[...]
    b = jnp.dot(x, we1_ref[...], preferred_element_type=jnp.float32)
    p = jnp.dot(x, wn0_ref[...], preferred_element_type=jnp.float32) + bn_ref[...]
    a2_ref[0] = a[:, :F]
    a2_ref[1] = a[:, F:]
    b2_ref[0] = b[:, :F]
    b2_ref[1] = b[:, F:]
    p_ref[...] = p


def _stage1(nf, we0, we1, be, wn0, bn):
    return pl.pallas_call(
        _stage1_body,
        grid=(GRID,),
        in_specs=[
            pl.BlockSpec((RB, D), lambda i: (i, 0)),
            pl.BlockSpec((D, D), lambda i: (0, 0)),
            pl.BlockSpec((D, D), lambda i: (0, 0)),
            pl.BlockSpec((1, D), lambda i: (0, 0)),
            pl.BlockSpec((D, D), lambda i: (0, 0)),
            pl.BlockSpec((1, D), lambda i: (0, 0)),
        ],
        out_specs=[
            pl.BlockSpec((NC, RB, F), lambda i: (0, i, 0)),
            pl.BlockSpec((NC, RB, F), lambda i: (0, i, 0)),
            pl.BlockSpec((RB, D), lambda i: (i, 0)),
        ],
        out_shape=[
            jax.ShapeDtypeStruct((NC, N_NODES, F), jnp.float32),
            jax.ShapeDtypeStruct((NC, N_NODES, F), jnp.float32),
            jax.ShapeDtypeStruct((N_NODES, D), jnp.float32),
        ],
    )(nf, we0, we1, be, wn0, bn)


def _edge_body(a2, b2, wt2, src_hbm, dst_hbm, traj_hbm,
               red2,
               src_v, dst_v, traj_v, a_v, b_v, wt_v, z_v,
               acc,
               sem_a, sem_b):
    c = lax.axis_index("c")
    s = lax.axis_index("s")

    zero = jnp.zeros((L,), jnp.float32)

    def _zero_row(e, _):
        for j in range(F // L):
            z_v[e, pl.ds(j * L, L)] = zero
        return 0

    lax.fori_loop(0, ROWS_PER_CP, _zero_row, 0)

    def _zero_acc(k, _):
        i = s + k * NS

        @pl.when(i < NROW_CHUNKS)
        def _():
            pltpu.sync_copy(z_v, acc.at[pl.ds(i * ROWS_PER_CP, ROWS_PER_CP)])

        return 0

    lax.fori_loop(0, NCP_ITERS, _zero_acc, 0)
    plsc.subcore_barrier()

    pltpu.sync_copy(wt2.at[c], wt_v)
    wt = [wt_v[pl.ds(j * L, L)] for j in range(F // L)]

    def _chunk(i, _):
        base = s * EPW + i * K
        pltpu.sync_copy(src_hbm.at[pl.ds(base, K)], src_v)
        pltpu.sync_copy(dst_hbm.at[pl.ds(base, K)], dst_v)
        pltpu.sync_copy(traj_hbm.at[pl.ds(base, K)], traj_v)
        cp_a = pltpu.async_copy(a2.at[c].at[src_v], a_v, sem_a)
        cp_b = pltpu.async_copy(b2.at[c].at[dst_v], b_v, sem_b)
        cp_a.wait()
        cp_b.wait()

        def _edge_grp(g, _):
            tvec = traj_v[pl.ds(g * L, L)]
            for e16 in range(L):
                e = g * L + e16
                t = tvec[e16]
                for j in range(F // L):
                    x = (a_v[e, pl.ds(j * L, L)] + b_v[e, pl.ds(j * L, L)]
                         + t * wt[j])
                    a_v[e, pl.ds(j * L, L)] = jnp.where(x > 0.0, x, x * 0.01)
            return 0

        lax.fori_loop(0, K // L, _edge_grp, 0)
        pltpu.sync_copy(a_v, acc.at[dst_v], add=True)
        return 0

    lax.fori_loop(0, NCHUNK, _chunk, 0)
    plsc.subcore_barrier()

    def _out(k, _):
        i = s + k * NS

        @pl.when(i < NROW_CHUNKS)
        def _():
            r0 = i * ROWS_PER_CP
            pltpu.sync_copy(acc.at[pl.ds(r0, ROWS_PER_CP)],
                            red2.at[c].at[pl.ds(r0, ROWS_PER_CP)])

        return 0

    lax.fori_loop(0, NCP_ITERS, _out, 0)


def _stage2(a2, b2, wt2, src, dst, traj):
    mesh = plsc.VectorSubcoreMesh(core_axis_name="c", subcore_axis_name="s")
    f = pl.kernel(
        _edge_body,
        out_type=jax.ShapeDtypeStruct((NC, N_NODES, F), jnp.float32),
        mesh=mesh,
        scratch_types=[
            pltpu.VMEM((K,), jnp.int32),
            pltpu.VMEM((K,), jnp.int32),
            pltpu.VMEM((K,), jnp.float32),
            pltpu.VMEM((K, F), jnp.float32),
            pltpu.VMEM((K, F), jnp.float32),
            pltpu.VMEM((F,), jnp.float32),
            pltpu.VMEM((ROWS_PER_CP, F), jnp.float32),
            pltpu.VMEM_SHARED((N_NODES, F), jnp.float32),
            pltpu.SemaphoreType.DMA,
            pltpu.SemaphoreType.DMA,
        ],
    )
    return f(a2, b2, wt2, src, dst, traj)


def _stage3_body(p_ref, red2_ref, wn1_ref, out_ref):
    r0 = red2_ref[0]
    r1 = red2_ref[1]
    y = (p_ref[...]
         + jnp.dot(r0, wn1_ref[0], preferred_element_type=jnp.float32)
         + jnp.dot(r1, wn1_ref[1], preferred_element_type=jnp.float32))
    out_ref[...] = jnp.where(y > 0.0, y, y * 0.01)


def _stage3(p, red2, wn1):
    return pl.pallas_call(
        _stage3_body,
        grid=(GRID,),
        in_specs=[
            pl.BlockSpec((RB, D), lambda i: (i, 0)),
            pl.BlockSpec((NC, RB, F), lambda i: (0, i, 0)),
            pl.BlockSpec((NC, F, D), lambda i: (0, 0, 0)),
        ],
        out_specs=pl.BlockSpec((RB, D), lambda i: (i, 0)),
        out_shape=jax.ShapeDtypeStruct((N_NODES, D), jnp.float32),
    )(p, red2, wn1)


@jax.jit
def kernel(nf, edge_index, traj, W_e, b_e, W_n, b_n):
    src = edge_index[0].astype(jnp.int32)
    dst = edge_index[1].astype(jnp.int32)
    we0 = W_e[:D]
    we1 = W_e[D:2 * D]
    wt2 = W_e[2 * D].reshape(NC, F)
    wn0 = W_n[:D]
    wn1 = W_n[D:].reshape(NC, F, D)
    a2, b2, p = _stage1(nf, we0, we1, b_e.reshape(1, D), wn0, b_n.reshape(1, D))
    red2 = _stage2(a2, b2, wt2, src, dst, traj)
    return _stage3(p, red2, wn1)


# speedup vs baseline: 3.8046x; 1.2401x over previous
import functools

import jax
import jax.numpy as jnp
from jax import lax
from jax.experimental import pallas as pl
from jax.experimental.pallas import tpu as pltpu
from jax.experimental.pallas import tpu_sc as plsc

N_NODES = 10000
N_EDGES = 160000
D = 256
F = 128
NC = 2
NS = 16
L = 16
E_PAD = 163840
EPW = E_PAD // NS
K = 32
NCHUNK = EPW // K
NB = 4
IB = NCHUNK // NB
EQ = IB * K
RQ = EQ // F
A_ROWS = N_NODES + NS
ROWS_PER_CP = 80
NROW_CHUNKS = N_NODES // ROWS_PER_CP
NCP_ITERS = -(-NROW_CHUNKS // NS)
NGRP = K // L

RB = 1000
GRID = N_NODES // RB


def _stage1_body(nf_ref, we0_ref, we1_ref, be_ref, wn0_ref, bn_ref,
                 a2_ref, b2_ref, p_ref):
    x = nf_ref[...]
    a = jnp.dot(x, we0_ref[...], preferred_element_type=jnp.float32) + be_ref[...]
    b = jnp.dot(x, we1_ref[...], preferred_element_type=jnp.float32)
    p = jnp.dot(x, wn0_ref[...], preferred_element_type=jnp.float32) + bn_ref[...]
    a2_ref[0] = a[:, :F]
    a2_ref[1] = a[:, F:]
    b2_ref[0] = b[:, :F]
    b2_ref[1] = b[:, F:]
    p_ref[...] = p


def _stage1(nf, we0, we1, be, wn0, bn):
    return pl.pallas_call(
        _stage1_body,
        grid=(GRID,),
        in_specs=[
            pl.BlockSpec((RB, D), lambda i: (i, 0)),
            pl.BlockSpec((D, D), lambda i: (0, 0)),
            pl.BlockSpec((D, D), lambda i: (0, 0)),
            pl.BlockSpec((1, D), lambda i: (0, 0)),
            pl.BlockSpec((D, D), lambda i: (0, 0)),
            pl.BlockSpec((1, D), lambda i: (0, 0)),
        ],
        out_specs=[
            pl.BlockSpec((NC, RB, F), lambda i: (0, i, 0)),
            pl.BlockSpec((NC, RB, F), lambda i: (0, i, 0)),
            pl.BlockSpec((RB, D), lambda i: (i, 0)),
        ],
        out_shape=[
            jax.ShapeDtypeStruct((NC, N_NODES, F), jnp.float32),
            jax.ShapeDtypeStruct((NC, N_NODES, F), jnp.float32),
            jax.ShapeDtypeStruct((N_NODES, D), jnp.float32),
        ],
    )(nf, we0, we1, be, wn0, bn)


def _edge_body(a2, b2, wt2, src4, dst4, traj4,
               red2,
               src_v, dst_v, traj_v,
               a0, a1, b0, b1, m0, m1, wt_v,
               acc,
               ga0, ga1, gb0, gb1, sc0, sc1):
    c = lax.axis_index("c")
    s = lax.axis_index("s")
    a_bufs, b_bufs, m_bufs = (a0, a1), (b0, b1), (m0, m1)
    gsem_a, gsem_b, ssem = (ga0, ga1), (gb0, gb1), (sc0, sc1)

    pltpu.sync_copy(wt2.at[c], wt_v)

    zero = jnp.zeros((L,), jnp.float32)

    def _zero_row(e, _):
        for j in range(F // L):
            m0[e, pl.ds(j * L, L)] = zero
        return 0

    lax.fori_loop(0, K, _zero_row, 0)

    n_zch = A_ROWS // K

    def _zero_acc(k, _):
        i = s + k * NS

        @pl.when(i < n_zch)
        def _():
            pltpu.sync_copy(m0, acc.at[pl.ds(i * K, K)])

        return 0

    lax.fori_loop(0, -(-n_zch // NS), _zero_acc, 0)
    plsc.subcore_barrier()

    wt = [wt_v[pl.ds(j * L, L)] for j in range(F // L)]

    def _issue_gathers(i, p):
        sidx = src_v.at[i // NB, pl.ds((i % NB) * K, K)]
        pltpu.async_copy(a2.at[c].at[sidx], a_bufs[p], gsem_a[p])
        pltpu.async_copy(b2.at[c].at[dst_v.at[i]], b_bufs[p], gsem_b[p])

    def _wait_gathers(p):
        pltpu.make_async_copy(a2.at[c].at[pl.ds(0, K)], a_bufs[p], gsem_a[p]).wait()
        pltpu.make_async_copy(b2.at[c].at[pl.ds(0, K)], b_bufs[p], gsem_b[p]).wait()

    def _wait_scatter(p):
        pltpu.make_async_copy(a2.at[c].at[pl.ds(0, K)], m_bufs[p], ssem[p]).wait()

    def _compute_chunk(i, p):
        a_v, b_v, m_v = a_bufs[p], b_bufs[p], m_bufs[p]

        def _grp(g, _):
            tvec = traj_v[i // NB, pl.ds((i % NB) * K + g * L, L)]
            for e16 in range(L):
                e = g * L + e16
                t = tvec[e16]
                for j in range(F // L):
                    sl = pl.ds(j * L, L)
                    x = a_v[e, sl] + b_v[e, sl] + t * wt[j]
                    m_v[e, sl] = jnp.maximum(x, x * 0.01)
            return 0

        lax.fori_loop(0, NGRP, _grp, 0)

    def _quarter(h, _):
        pltpu.sync_copy(src4.at[s * NB + h], src_v)
        pltpu.sync_copy(dst4.at[s * NB + h], dst_v)
        pltpu.sync_copy(traj4.at[s * NB + h], traj_v)
        _issue_gathers(0, 0)
        _issue_gathers(1, 1)

        def _pair(k, _):
            for p in range(2):
                i = 2 * k + p
                _wait_gathers(p)

                @pl.when(i >= 2)
                def _():
                    _wait_scatter(p)

                _compute_chunk(i, p)
                pltpu.async_copy(m_bufs[p], acc.at[dst_v.at[i]], ssem[p],
                                 add=True)

                @pl.when(i + 2 < IB)
                def _():
                    _issue_gathers(i + 2, p)

            return 0

        lax.fori_loop(0, IB // 2, _pair, 0)
        _wait_scatter(0)
        _wait_scatter(1)
        return 0

    lax.fori_loop(0, NB, _quarter, 0)
    plsc.subcore_barrier()

    def _out(k, _):
        i = s + k * NS

        @pl.when(i < NROW_CHUNKS)
        def _():
            r0 = i * ROWS_PER_CP
            pltpu.sync_copy(acc.at[pl.ds(r0, ROWS_PER_CP)],
                            red2.at[c].at[pl.ds(r0, ROWS_PER_CP)])

        return 0

    lax.fori_loop(0, NCP_ITERS, _out, 0)


def _stage2(a2, b2, wt2, src4, dst4, traj4):
    mesh = plsc.VectorSubcoreMesh(core_axis_name="c", subcore_axis_name="s")
    f = pl.kernel(
        _edge_body,
        out_type=jax.ShapeDtypeStruct((NC, N_NODES, F), jnp.float32),
        mesh=mesh,
        scratch_types=[
            pltpu.VMEM((RQ, F), jnp.int32),
            pltpu.VMEM((IB, K), jnp.int32),
            pltpu.VMEM((RQ, F), jnp.float32),
            pltpu.VMEM((K, F), jnp.float32),
            pltpu.VMEM((K, F), jnp.float32),
            pltpu.VMEM((K, F), jnp.float32),
            pltpu.VMEM((K, F), jnp.float32),
            pltpu.VMEM((K, F), jnp.float32),
            pltpu.VMEM((K, F), jnp.float32),
            pltpu.VMEM((F,), jnp.float32),
            pltpu.VMEM_SHARED((A_ROWS, F), jnp.float32),
            pltpu.SemaphoreType.DMA,
            pltpu.SemaphoreType.DMA,
            pltpu.SemaphoreType.DMA,
            pltpu.SemaphoreType.DMA,
            pltpu.SemaphoreType.DMA,
            pltpu.SemaphoreType.DMA,
        ],
    )
    return f(a2, b2, wt2, src4, dst4, traj4)


def _stage3_body(p_ref, red2_ref, wn1_ref, out_ref):
    r0 = red2_ref[0]
    r1 = red2_ref[1]
    y = (p_ref[...]
         + jnp.dot(r0, wn1_ref[0], preferred_element_type=jnp.float32)
         + jnp.dot(r1, wn1_ref[1], preferred_element_type=jnp.float32))
    out_ref[...] = jnp.where(y > 0.0, y, y * 0.01)


def _stage3(p, red2, wn1):
    return pl.pallas_call(
        _stage3_body,
        grid=(GRID,),
        in_specs=[
            pl.BlockSpec((RB, D), lambda i: (i, 0)),
            pl.BlockSpec((NC, RB, F), lambda i: (0, i, 0)),
            pl.BlockSpec((NC, F, D), lambda i: (0, 0, 0)),
        ],
        out_specs=pl.BlockSpec((RB, D), lambda i: (i, 0)),
        out_shape=jax.ShapeDtypeStruct((N_NODES, D), jnp.float32),
    )(p, red2, wn1)


@jax.jit
def kernel(nf, edge_index, traj, W_e, b_e, W_n, b_n):
    src = edge_index[0].astype(jnp.int32)
    dst = edge_index[1].astype(jnp.int32)
    we0 = W_e[:D]
    we1 = W_e[D:2 * D]
    wt2 = W_e[2 * D].reshape(NC, F)
    wn0 = W_n[:D]
    wn1 = W_n[D:].reshape(NC, F, D)
    pad = E_PAD - N_EDGES
    src = jnp.concatenate([src, jnp.zeros((pad,), jnp.int32)])
    dst = jnp.concatenate(
        [dst, N_NODES + (jnp.arange(pad, dtype=jnp.int32) % NS)])
    traj = jnp.concatenate([traj, jnp.zeros((pad,), jnp.float32)])
    src4 = src.reshape(NS * NB, RQ, F)
    dst4 = dst.reshape(NS * NB, IB, K)
    traj4 = traj.reshape(NS * NB, RQ, F)
    a2, b2, p = _stage1(nf, we0, we1, b_e.reshape(1, D), wn0, b_n.reshape(1, D))
    red2 = _stage2(a2, b2, wt2, src4, dst4, traj4)
    return _stage3(p, red2, wn1)
